# baseline (speedup 1.0000x reference)
import functools

import jax
import jax.numpy as jnp
from jax import lax
from jax.experimental import pallas as pl
from jax.experimental.pallas import tpu as pltpu
from jax.experimental.pallas import tpu_sc as plsc

_NUM_VQ = 512
_NUM_RVQ = 512
_CLIP = 512
_D = 1024
_NQ = 6
_BS = 64
_T = 64

_QM1 = _NQ - 1
_CB = _NUM_RVQ + 2
_ROWS = _QM1 * _CB
_ZROW = _ROWS
_TOK = _BS * _T
_NW = 32
_TPW = _TOK // _NW
_G = 8
_RPC = _G * _QM1
_NCHUNK = _TPW // _G
_NIDX = _TPW * _QM1


def _sc_hist_body(table, rfl, qids, hist, r_v, q_v, idx_v, rows_v, rows_v1,
                  out_v, out_v1, sem):
    wid = lax.axis_index("s") * 2 + lax.axis_index("c")
    base = wid * _TPW

    pltpu.sync_copy(rfl.at[0], r_v)
    pltpu.sync_copy(qids, q_v)

    def idx_step(c, carry):
        p = c * 16 + lax.iota(jnp.int32, 16)
        ch = p // _RPC
        g = p - ch * _RPC
        qq = g >> 3
        t_loc = ch * _G + (g & (_G - 1))
        b = t_loc >> 6
        rv = plsc.load_gather(r_v, [t_loc, qq])
        qid = plsc.load_gather(q_v, [b])
        n = lax.rem(qid + (_QM1 - 1), _QM1)
        flat = jnp.where(qq <= n, qq * _CB + rv, _ZROW)
        idx_v[pl.ds(c * 16, 16)] = flat
        return carry

    lax.fori_loop(0, _NIDX // 16, idx_step, 0)

    def start_gather(c, buf):
        off = pl.multiple_of(c * _RPC, 8)
        return pltpu.async_copy(table.at[idx_v.at[pl.ds(off, _RPC)]], buf, sem)

    def wait_gather(buf):
        pltpu.make_async_copy(table.at[idx_v.at[pl.ds(0, _RPC)]], buf, sem).wait()

    def accum(rows, out_ref):
        def dstep(dd, carry2):
            sl = pl.ds(dd * 16, 16)
            for t in range(_G):
                s = rows[0 * _G + t, sl] + rows[1 * _G + t, sl]
                s = s + rows[2 * _G + t, sl]
                s = s + rows[3 * _G + t, sl]
                s = s + rows[4 * _G + t, sl]
                out_ref[t, sl] = s
            return carry2
        lax.fori_loop(0, _D // 16, dstep, 0)

    start_gather(0, rows_v)

    def pair_step(i, carry):
        c0 = i * 2
        start_gather(c0 + 1, rows_v1)
        wait_gather(rows_v)
        accum(rows_v, out_v)
        pltpu.sync_copy(out_v, hist.at[pl.ds(base + c0 * _G, _G)])

        @pl.when(i < _NCHUNK // 2 - 1)
        def _():
            start_gather(c0 + 2, rows_v)
        wait_gather(rows_v1)
        accum(rows_v1, out_v1)
        pltpu.sync_copy(out_v1, hist.at[pl.ds(base + (c0 + 1) * _G, _G)])
        return carry

    lax.fori_loop(0, _NCHUNK // 2, pair_step, 0)


_sc_hist = pl.kernel(
    _sc_hist_body,
    out_type=jax.ShapeDtypeStruct((_TOK, _D), jnp.float32),
    compiler_params=pltpu.CompilerParams(needs_layout_passes=False),
    mesh=plsc.VectorSubcoreMesh(core_axis_name="c", subcore_axis_name="s",
                                num_cores=2, num_subcores=16),
    scratch_types=[
        pltpu.VMEM((_TPW, _NQ), jnp.int32),
        pltpu.VMEM((_BS,), jnp.int32),
        pltpu.VMEM((_NIDX,), jnp.int32),
        pltpu.VMEM((_RPC, _D), jnp.float32),
        pltpu.VMEM((_RPC, _D), jnp.float32),
        pltpu.VMEM((_G, _D), jnp.float32),
        pltpu.VMEM((_G, _D), jnp.float32),
        pltpu.SemaphoreType.DMA,
    ],
)

_MT = 256


def _tc_body(p_ref, wpose_ref, bpose_ref, clip_ref, wcond_ref, bcond_ref,
             qi_ref, wq_ref, bq_ref, pemb_ref, cond_ref, qemb_ref):
    pemb_ref[...] = (
        jnp.dot(p_ref[...], wpose_ref[...], preferred_element_type=jnp.float32)
        + bpose_ref[...]
    )

    @pl.when(pl.program_id(0) == 0)
    def _():
        cond_ref[...] = (
            jnp.dot(clip_ref[...], wcond_ref[...],
                    preferred_element_type=jnp.float32)
            + bcond_ref[...]
        )
        ioq = lax.broadcasted_iota(jnp.int32, (_BS, _NQ), 1)
        oh = (ioq == qi_ref[...]).astype(jnp.float32)
        qemb_ref[...] = (
            jnp.dot(oh, wq_ref[...], preferred_element_type=jnp.float32)
            + bq_ref[...]
        )


_tc_call = pl.pallas_call(
    _tc_body,
    grid=(_TOK // _MT,),
    in_specs=[
        pl.BlockSpec((_MT, _NUM_VQ + 2), lambda i: (i, 0)),
        pl.BlockSpec((_NUM_VQ + 2, _D), lambda i: (0, 0)),
        pl.BlockSpec((1, _D), lambda i: (0, 0)),
        pl.BlockSpec((_BS, _CLIP), lambda i: (0, 0)),
        pl.BlockSpec((_CLIP, _D), lambda i: (0, 0)),
        pl.BlockSpec((1, _D), lambda i: (0, 0)),
        pl.BlockSpec((_BS, 1), lambda i: (0, 0)),
        pl.BlockSpec((_NQ, _D), lambda i: (0, 0)),
        pl.BlockSpec((1, _D), lambda i: (0, 0)),
    ],
    out_specs=[
        pl.BlockSpec((_MT, _D), lambda i: (i, 0)),
        pl.BlockSpec((_BS, _D), lambda i: (0, 0)),
        pl.BlockSpec((_BS, _D), lambda i: (0, 0)),
    ],
    out_shape=[
        jax.ShapeDtypeStruct((_TOK, _D), jnp.float32),
        jax.ShapeDtypeStruct((_BS, _D), jnp.float32),
        jax.ShapeDtypeStruct((_BS, _D), jnp.float32),
    ],
)


def kernel(clip_feature, q_ids, p_codes, r_codes, W_cond, b_cond, W_pose,
           b_pose, W_quant, b_quant, token_embed_weight):
    qi = q_ids.astype(jnp.int32)
    table = jnp.concatenate(
        [token_embed_weight.reshape(_ROWS, _D),
         jnp.zeros((1, _D), jnp.float32)], axis=0)
    rfl = r_codes.astype(jnp.int32).reshape(_NW, _TPW, _NQ)

    hist = _sc_hist(table, rfl, qi)
    pemb, cond, qemb = _tc_call(
        p_codes.reshape(_TOK, _NUM_VQ + 2), W_pose, b_pose.reshape(1, _D),
        clip_feature, W_cond, b_cond.reshape(1, _D),
        qi.reshape(_BS, 1), W_quant, b_quant.reshape(1, _D))

    return (cond, qemb, pemb.reshape(_BS, _T, _D), hist.reshape(_BS, _T, _D))


# speedup vs baseline: 2.4386x; 2.4386x over previous
import functools

import jax
import jax.numpy as jnp
from jax import lax
from jax.experimental import pallas as pl
from jax.experimental.pallas import tpu as pltpu
from jax.experimental.pallas import tpu_sc as plsc

_NUM_VQ = 512
_NUM_RVQ = 512
_CLIP = 512
_D = 1024
_NQ = 6
_BS = 64
_T = 64

_QM1 = _NQ - 1
_CB = _NUM_RVQ + 2
_ROWS = _QM1 * _CB
_ZROW = _ROWS
_TOK = _BS * _T
_F = 3072
_SCT = _TOK - _F
_NW = 32
_TPW = _SCT // _NW
_G = 8
_RPC = _G * _QM1
_NCHUNK = _TPW // _G
_NIDX = _TPW * _QM1


def _sc_hist_body(table, rfl, qids, hist, r_v, q_v, idx_v, rows_v, rows_v1,
                  out_v, out_v1, sem):
    wid = lax.axis_index("s") * 2 + lax.axis_index("c")
    base = wid * _TPW

    pltpu.sync_copy(rfl.at[wid], r_v)
    pltpu.sync_copy(qids, q_v)

    def idx_step(c, carry):
        p = c * 16 + lax.iota(jnp.int32, 16)
        ch = p // _RPC
        g = p - ch * _RPC
        qq = g >> 3
        t_loc = ch * _G + (g & (_G - 1))
        b = (_F + base + t_loc) >> 6
        rv = plsc.load_gather(r_v, [t_loc, qq])
        qid = plsc.load_gather(q_v, [b])
        n = lax.rem(qid + (_QM1 - 1), _QM1)
        flat = jnp.where(qq <= n, qq * _CB + rv, _ZROW)
        idx_v[pl.ds(c * 16, 16)] = flat
        return carry

    lax.fori_loop(0, _NIDX // 16, idx_step, 0)

    def start_gather(c, buf):
        off = pl.multiple_of(c * _RPC, 8)
        return pltpu.async_copy(table.at[idx_v.at[pl.ds(off, _RPC)]], buf, sem)

    def wait_gather(buf):
        pltpu.make_async_copy(table.at[idx_v.at[pl.ds(0, _RPC)]], buf, sem).wait()

    def accum(rows, out_ref):
        def dstep(dd, carry2):
            sl = pl.ds(dd * 16, 16)
            for t in range(_G):
                s = rows[0 * _G + t, sl] + rows[1 * _G + t, sl]
                s = s + rows[2 * _G + t, sl]
                s = s + rows[3 * _G + t, sl]
                s = s + rows[4 * _G + t, sl]
                out_ref[t, sl] = s
            return carry2
        lax.fori_loop(0, _D // 16, dstep, 0)

    start_gather(0, rows_v)

    def pair_step(i, carry):
        c0 = i * 2
        start_gather(c0 + 1, rows_v1)
        wait_gather(rows_v)
        accum(rows_v, out_v)
        pltpu.sync_copy(out_v, hist.at[pl.ds(base + c0 * _G, _G)])

        @pl.when(i < _NCHUNK // 2 - 1)
        def _():
            start_gather(c0 + 2, rows_v)
        wait_gather(rows_v1)
        accum(rows_v1, out_v1)
        pltpu.sync_copy(out_v1, hist.at[pl.ds(base + (c0 + 1) * _G, _G)])
        return carry

    lax.fori_loop(0, _NCHUNK // 2, pair_step, 0)


_sc_hist = pl.kernel(
    _sc_hist_body,
    out_type=jax.ShapeDtypeStruct((_SCT, _D), jnp.float32),
    compiler_params=pltpu.CompilerParams(needs_layout_passes=False),
    mesh=plsc.VectorSubcoreMesh(core_axis_name="c", subcore_axis_name="s",
                                num_cores=2, num_subcores=16),
    scratch_types=[
        pltpu.VMEM((_TPW, _NQ), jnp.int32),
        pltpu.VMEM((_BS,), jnp.int32),
        pltpu.VMEM((_NIDX,), jnp.int32),
        pltpu.VMEM((_RPC, _D), jnp.float32),
        pltpu.VMEM((_RPC, _D), jnp.float32),
        pltpu.VMEM((_G, _D), jnp.float32),
        pltpu.VMEM((_G, _D), jnp.float32),
        pltpu.SemaphoreType.DMA,
    ],
)

_MT = 256


def _tc_hist_body(r_ref, qn_ref, tab_ref, out_ref):
    acc = None
    col = lax.broadcasted_iota(jnp.int32, (_MT, _CB), 1)
    n = qn_ref[...]
    for q in range(_QM1):
        rq = r_ref[:, q:q + 1]
        oh = ((col == rq).astype(jnp.float32)
              * (q <= n).astype(jnp.float32)).astype(jnp.bfloat16)
        part = jnp.dot(oh, tab_ref[pl.ds(q * _CB, _CB), :],
                       preferred_element_type=jnp.float32)
        acc = part if acc is None else acc + part
    out_ref[...] = acc


_tc_hist_call = pl.pallas_call(
    _tc_hist_body,
    grid=(_F // _MT,),
    in_specs=[
        pl.BlockSpec((_MT, _NQ), lambda i: (i, 0)),
        pl.BlockSpec((_MT, 1), lambda i: (i, 0)),
        pl.BlockSpec((_ROWS, _D), lambda i: (0, 0)),
    ],
    out_specs=pl.BlockSpec((_MT, _D), lambda i: (i, 0)),
    out_shape=jax.ShapeDtypeStruct((_F, _D), jnp.float32),
)


def _tc_body(p_ref, wpose_ref, bpose_ref, clip_ref, wcond_ref, bcond_ref,
             qi_ref, wq_ref, bq_ref, pemb_ref, cond_ref, qemb_ref):
    pemb_ref[...] = (
        jnp.dot(p_ref[...], wpose_ref[...], preferred_element_type=jnp.float32)
        + bpose_ref[...]
    )

    @pl.when(pl.program_id(0) == 0)
    def _():
        cond_ref[...] = (
            jnp.dot(clip_ref[...], wcond_ref[...],
                    preferred_element_type=jnp.float32)
            + bcond_ref[...]
        )
        ioq = lax.broadcasted_iota(jnp.int32, (_BS, _NQ), 1)
        oh = (ioq == qi_ref[...]).astype(jnp.float32)
        qemb_ref[...] = (
            jnp.dot(oh, wq_ref[...], preferred_element_type=jnp.float32)
            + bq_ref[...]
        )


_tc_call = pl.pallas_call(
    _tc_body,
    grid=(_TOK // _MT,),
    in_specs=[
        pl.BlockSpec((_MT, _NUM_VQ + 2), lambda i: (i, 0)),
        pl.BlockSpec((_NUM_VQ + 2, _D), lambda i: (0, 0)),
        pl.BlockSpec((1, _D), lambda i: (0, 0)),
        pl.BlockSpec((_BS, _CLIP), lambda i: (0, 0)),
        pl.BlockSpec((_CLIP, _D), lambda i: (0, 0)),
        pl.BlockSpec((1, _D), lambda i: (0, 0)),
        pl.BlockSpec((_BS, 1), lambda i: (0, 0)),
        pl.BlockSpec((_NQ, _D), lambda i: (0, 0)),
        pl.BlockSpec((1, _D), lambda i: (0, 0)),
    ],
    out_specs=[
        pl.BlockSpec((_MT, _D), lambda i: (i, 0)),
        pl.BlockSpec((_BS, _D), lambda i: (0, 0)),
        pl.BlockSpec((_BS, _D), lambda i: (0, 0)),
    ],
    out_shape=[
        jax.ShapeDtypeStruct((_TOK, _D), jnp.float32),
        jax.ShapeDtypeStruct((_BS, _D), jnp.float32),
        jax.ShapeDtypeStruct((_BS, _D), jnp.float32),
    ],
)


def kernel(clip_feature, q_ids, p_codes, r_codes, W_cond, b_cond, W_pose,
           b_pose, W_quant, b_quant, token_embed_weight):
    qi = q_ids.astype(jnp.int32)
    table = jnp.concatenate(
        [token_embed_weight.reshape(_ROWS, _D),
         jnp.zeros((1, _D), jnp.float32)], axis=0)
    r2 = r_codes.astype(jnp.int32).reshape(_TOK, _NQ)
    rfl = r2[_F:].reshape(_NW, _TPW, _NQ)

    hist_sc = _sc_hist(table, rfl, qi)

    tab16 = token_embed_weight.reshape(_ROWS, _D).astype(jnp.bfloat16)
    qn_tok = lax.rem(jnp.repeat(qi, _T) + (_QM1 - 1), _QM1)[:_F].reshape(_F, 1)
    hist_tc = _tc_hist_call(r2[:_F], qn_tok, tab16)
    hist = jnp.concatenate([hist_tc, hist_sc], axis=0)
    pemb, cond, qemb = _tc_call(
        p_codes.reshape(_TOK, _NUM_VQ + 2), W_pose, b_pose.reshape(1, _D),
        clip_feature, W_cond, b_cond.reshape(1, _D),
        qi.reshape(_BS, 1), W_quant, b_quant.reshape(1, _D))

    return (cond, qemb, pemb.reshape(_BS, _T, _D), hist.reshape(_BS, _T, _D))


# speedup vs baseline: 3.0247x; 1.2404x over previous
import functools

import jax
import jax.numpy as jnp
from jax import lax
from jax.experimental import pallas as pl
from jax.experimental.pallas import tpu as pltpu
from jax.experimental.pallas import tpu_sc as plsc

_NUM_VQ = 512
_NUM_RVQ = 512
_CLIP = 512
_D = 1024
_NQ = 6
_BS = 64
_T = 64

_QM1 = _NQ - 1
_CB = _NUM_RVQ + 2
_ROWS = _QM1 * _CB
_ZROW = _ROWS
_TOK = _BS * _T
_F = 3584
_SCT = _TOK - _F
_NW = 32
_TPW = _SCT // _NW
_G = 8
_RPC = _G * _QM1
_NCHUNK = _TPW // _G
_NIDX = _TPW * _QM1


def _sc_hist_body(table, rfl, qids, hist, r_v, q_v, idx_v, rows_v, rows_v1,
                  out_v, out_v1, sem):
    wid = lax.axis_index("s") * 2 + lax.axis_index("c")
    base = wid * _TPW

    pltpu.sync_copy(rfl.at[wid], r_v)
    pltpu.sync_copy(qids, q_v)

    def idx_step(c, carry):
        p = c * 16 + lax.iota(jnp.int32, 16)
        ch = p // _RPC
        g = p - ch * _RPC
        qq = g >> 3
        t_loc = ch * _G + (g & (_G - 1))
        b = (_F + base + t_loc) >> 6
        rv = plsc.load_gather(r_v, [t_loc, qq])
        qid = plsc.load_gather(q_v, [b])
        n = lax.rem(qid + (_QM1 - 1), _QM1)
        flat = jnp.where(qq <= n, qq * _CB + rv, _ZROW)
        idx_v[pl.ds(c * 16, 16)] = flat
        return carry

    lax.fori_loop(0, _NIDX // 16, idx_step, 0)

    def start_gather(c, buf):
        off = pl.multiple_of(c * _RPC, 8)
        return pltpu.async_copy(table.at[idx_v.at[pl.ds(off, _RPC)]], buf, sem)

    def wait_gather(buf):
        pltpu.make_async_copy(table.at[idx_v.at[pl.ds(0, _RPC)]], buf, sem).wait()

    def accum(rows, out_ref):
        def dstep(dd, carry2):
            sl = pl.ds(dd * 16, 16)
            for t in range(_G):
                s = rows[0 * _G + t, sl] + rows[1 * _G + t, sl]
                s = s + rows[2 * _G + t, sl]
                s = s + rows[3 * _G + t, sl]
                s = s + rows[4 * _G + t, sl]
                out_ref[t, sl] = s
            return carry2
        lax.fori_loop(0, _D // 16, dstep, 0)

    start_gather(0, rows_v)

    def pair_step(i, carry):
        c0 = i * 2
        start_gather(c0 + 1, rows_v1)
        wait_gather(rows_v)
        accum(rows_v, out_v)
        pltpu.sync_copy(out_v, hist.at[pl.ds(base + c0 * _G, _G)])

        @pl.when(i < _NCHUNK // 2 - 1)
        def _():
            start_gather(c0 + 2, rows_v)
        wait_gather(rows_v1)
        accum(rows_v1, out_v1)
        pltpu.sync_copy(out_v1, hist.at[pl.ds(base + (c0 + 1) * _G, _G)])
        return carry

    lax.fori_loop(0, _NCHUNK // 2, pair_step, 0)


_sc_hist = pl.kernel(
    _sc_hist_body,
    out_type=jax.ShapeDtypeStruct((_SCT, _D), jnp.float32),
    compiler_params=pltpu.CompilerParams(needs_layout_passes=False),
    mesh=plsc.VectorSubcoreMesh(core_axis_name="c", subcore_axis_name="s",
                                num_cores=2, num_subcores=16),
    scratch_types=[
        pltpu.VMEM((_TPW, _NQ), jnp.int32),
        pltpu.VMEM((_BS,), jnp.int32),
        pltpu.VMEM((_NIDX,), jnp.int32),
        pltpu.VMEM((_RPC, _D), jnp.float32),
        pltpu.VMEM((_RPC, _D), jnp.float32),
        pltpu.VMEM((_G, _D), jnp.float32),
        pltpu.VMEM((_G, _D), jnp.float32),
        pltpu.SemaphoreType.DMA,
    ],
)

_MT = 256


def _tc_hist_body(r_ref, qn_ref, tab_ref, out_ref):
    acc = None
    col = lax.broadcasted_iota(jnp.int32, (_MT, _CB), 1)
    n = qn_ref[...]
    for q in range(_QM1):
        rq = r_ref[:, q:q + 1]
        oh = ((col == rq).astype(jnp.float32)
              * (q <= n).astype(jnp.float32)).astype(jnp.bfloat16)
        part = jnp.dot(oh, tab_ref[pl.ds(q * _CB, _CB), :],
                       preferred_element_type=jnp.float32)
        acc = part if acc is None else acc + part
    out_ref[...] = acc


_tc_hist_call = pl.pallas_call(
    _tc_hist_body,
    grid=(_F // _MT,),
    in_specs=[
        pl.BlockSpec((_MT, _NQ), lambda i: (i, 0)),
        pl.BlockSpec((_MT, 1), lambda i: (i, 0)),
        pl.BlockSpec((_ROWS, _D), lambda i: (0, 0)),
    ],
    out_specs=pl.BlockSpec((_MT, _D), lambda i: (i, 0)),
    out_shape=jax.ShapeDtypeStruct((_F, _D), jnp.float32),
)


def _tc_body(p_ref, wpose_ref, bpose_ref, clip_ref, wcond_ref, bcond_ref,
             qi_ref, wq_ref, bq_ref, pemb_ref, cond_ref, qemb_ref):
    pemb_ref[...] = (
        jnp.dot(p_ref[...], wpose_ref[...], preferred_element_type=jnp.float32)
        + bpose_ref[...]
    )

    @pl.when(pl.program_id(0) == 0)
    def _():
        cond_ref[...] = (
            jnp.dot(clip_ref[...], wcond_ref[...],
                    preferred_element_type=jnp.float32)
            + bcond_ref[...]
        )
        ioq = lax.broadcasted_iota(jnp.int32, (_BS, _NQ), 1)
        oh = (ioq == qi_ref[...]).astype(jnp.float32)
        qemb_ref[...] = (
            jnp.dot(oh, wq_ref[...], preferred_element_type=jnp.float32)
            + bq_ref[...]
        )


_tc_call = pl.pallas_call(
    _tc_body,
    grid=(_TOK // _MT,),
    in_specs=[
        pl.BlockSpec((_MT, _NUM_VQ + 2), lambda i: (i, 0)),
        pl.BlockSpec((_NUM_VQ + 2, _D), lambda i: (0, 0)),
        pl.BlockSpec((1, _D), lambda i: (0, 0)),
        pl.BlockSpec((_BS, _CLIP), lambda i: (0, 0)),
        pl.BlockSpec((_CLIP, _D), lambda i: (0, 0)),
        pl.BlockSpec((1, _D), lambda i: (0, 0)),
        pl.BlockSpec((_BS, 1), lambda i: (0, 0)),
        pl.BlockSpec((_NQ, _D), lambda i: (0, 0)),
        pl.BlockSpec((1, _D), lambda i: (0, 0)),
    ],
    out_specs=[
        pl.BlockSpec((_MT, _D), lambda i: (i, 0)),
        pl.BlockSpec((_BS, _D), lambda i: (0, 0)),
        pl.BlockSpec((_BS, _D), lambda i: (0, 0)),
    ],
    out_shape=[
        jax.ShapeDtypeStruct((_TOK, _D), jnp.float32),
        jax.ShapeDtypeStruct((_BS, _D), jnp.float32),
        jax.ShapeDtypeStruct((_BS, _D), jnp.float32),
    ],
)


def kernel(clip_feature, q_ids, p_codes, r_codes, W_cond, b_cond, W_pose,
           b_pose, W_quant, b_quant, token_embed_weight):
    qi = q_ids.astype(jnp.int32)
    table = jnp.concatenate(
        [token_embed_weight.reshape(_ROWS, _D),
         jnp.zeros((1, _D), jnp.float32)], axis=0)
    r2 = r_codes.astype(jnp.int32).reshape(_TOK, _NQ)
    rfl = r2[_F:].reshape(_NW, _TPW, _NQ)

    hist_sc = _sc_hist(table, rfl, qi)

    tab16 = token_embed_weight.reshape(_ROWS, _D).astype(jnp.bfloat16)
    qn_tok = lax.rem(jnp.repeat(qi, _T) + (_QM1 - 1), _QM1)[:_F].reshape(_F, 1)
    hist_tc = _tc_hist_call(r2[:_F], qn_tok, tab16)
    hist = jnp.concatenate([hist_tc, hist_sc], axis=0)
    pemb, cond, qemb = _tc_call(
        p_codes.reshape(_TOK, _NUM_VQ + 2), W_pose, b_pose.reshape(1, _D),
        clip_feature, W_cond, b_cond.reshape(1, _D),
        qi.reshape(_BS, 1), W_quant, b_quant.reshape(1, _D))

    return (cond, qemb, pemb.reshape(_BS, _T, _D), hist.reshape(_BS, _T, _D))


# speedup vs baseline: 3.1046x; 1.0264x over previous
import functools

import jax
import jax.numpy as jnp
from jax import lax
from jax.experimental import pallas as pl
from jax.experimental.pallas import tpu as pltpu
from jax.experimental.pallas import tpu_sc as plsc

_NUM_VQ = 512
_NUM_RVQ = 512
_CLIP = 512
_D = 1024
_NQ = 6
_BS = 64
_T = 64

_QM1 = _NQ - 1
_CB = _NUM_RVQ + 2
_ROWS = _QM1 * _CB
_ZROW = _ROWS
_TOK = _BS * _T
_F = 3584
_SCT = _TOK - _F
_NW = 32
_TPW = _SCT // _NW
_G = 8
_RPC = _G * _QM1
_NCHUNK = _TPW // _G
_NIDX = _TPW * _QM1


def _sc_hist_body(table, rfl, qids, hist, r_v, q_v, idx_v, rows_v, rows_v1,
                  out_v, out_v1, sem):
    wid = lax.axis_index("s") * 2 + lax.axis_index("c")
    base = wid * _TPW

    pltpu.sync_copy(rfl.at[wid], r_v)
    pltpu.sync_copy(qids, q_v)

    def idx_step(c, carry):
        p = c * 16 + lax.iota(jnp.int32, 16)
        ch = p // _RPC
        g = p - ch * _RPC
        qq = g >> 3
        t_loc = ch * _G + (g & (_G - 1))
        b = (_F + base + t_loc) >> 6
        rv = plsc.load_gather(r_v, [t_loc, qq])
        qid = plsc.load_gather(q_v, [b])
        n = lax.rem(qid + (_QM1 - 1), _QM1)
        flat = jnp.where(qq <= n, qq * _CB + rv, _ZROW)
        idx_v[pl.ds(c * 16, 16)] = flat
        return carry

    lax.fori_loop(0, _NIDX // 16, idx_step, 0)

    def start_gather(c, buf):
        off = pl.multiple_of(c * _RPC, 8)
        return pltpu.async_copy(table.at[idx_v.at[pl.ds(off, _RPC)]], buf, sem)

    def wait_gather(buf):
        pltpu.make_async_copy(table.at[idx_v.at[pl.ds(0, _RPC)]], buf, sem).wait()

    def accum(rows, out_ref):
        def dstep(dd, carry2):
            sl = pl.ds(dd * 16, 16)
            for t in range(_G):
                s = rows[0 * _G + t, sl] + rows[1 * _G + t, sl]
                s = s + rows[2 * _G + t, sl]
                s = s + rows[3 * _G + t, sl]
                s = s + rows[4 * _G + t, sl]
                out_ref[t, sl] = s
            return carry2
        lax.fori_loop(0, _D // 16, dstep, 0)

    start_gather(0, rows_v)

    def pair_step(i, carry):
        c0 = i * 2
        start_gather(c0 + 1, rows_v1)
        wait_gather(rows_v)
        accum(rows_v, out_v)
        pltpu.sync_copy(out_v, hist.at[pl.ds(base + c0 * _G, _G)])

        @pl.when(i < _NCHUNK // 2 - 1)
        def _():
            start_gather(c0 + 2, rows_v)
        wait_gather(rows_v1)
        accum(rows_v1, out_v1)
        pltpu.sync_copy(out_v1, hist.at[pl.ds(base + (c0 + 1) * _G, _G)])
        return carry

    lax.fori_loop(0, _NCHUNK // 2, pair_step, 0)


_sc_hist = pl.kernel(
    _sc_hist_body,
    out_type=jax.ShapeDtypeStruct((_SCT, _D), jnp.float32),
    compiler_params=pltpu.CompilerParams(needs_layout_passes=False),
    mesh=plsc.VectorSubcoreMesh(core_axis_name="c", subcore_axis_name="s",
                                num_cores=2, num_subcores=16),
    scratch_types=[
        pltpu.VMEM((_TPW, _NQ), jnp.int32),
        pltpu.VMEM((_BS,), jnp.int32),
        pltpu.VMEM((_NIDX,), jnp.int32),
        pltpu.VMEM((_RPC, _D), jnp.float32),
        pltpu.VMEM((_RPC, _D), jnp.float32),
        pltpu.VMEM((_G, _D), jnp.float32),
        pltpu.VMEM((_G, _D), jnp.float32),
        pltpu.SemaphoreType.DMA,
    ],
)

_MT = 256


def _tc_body(p_ref, wpose_ref, bpose_ref, clip_ref, wcond_ref, bcond_ref,
             qi_ref, wq_ref, bq_ref, r_ref, qn_ref, tab_ref,
             pemb_ref, cond_ref, qemb_ref, htc_ref):
    pemb_ref[...] = (
        jnp.dot(p_ref[...], wpose_ref[...], preferred_element_type=jnp.float32)
        + bpose_ref[...]
    )

    @pl.when(pl.program_id(0) < _F // _MT)
    def _():
        acc = None
        col = lax.broadcasted_iota(jnp.int32, (_MT, _CB), 1)
        n = qn_ref[...]
        for q in range(_QM1):
            rq = r_ref[:, q:q + 1]
            oh = ((col == rq).astype(jnp.float32)
                  * (q <= n).astype(jnp.float32)).astype(jnp.bfloat16)
            part = jnp.dot(oh, tab_ref[pl.ds(q * _CB, _CB), :],
                           preferred_element_type=jnp.float32)
            acc = part if acc is None else acc + part
        htc_ref[...] = acc

    @pl.when(pl.program_id(0) == 0)
    def _():
        cond_ref[...] = (
            jnp.dot(clip_ref[...], wcond_ref[...],
                    preferred_element_type=jnp.float32)
            + bcond_ref[...]
        )
        ioq = lax.broadcasted_iota(jnp.int32, (_BS, _NQ), 1)
        oh = (ioq == qi_ref[...]).astype(jnp.float32)
        qemb_ref[...] = (
            jnp.dot(oh, wq_ref[...], preferred_element_type=jnp.float32)
            + bq_ref[...]
        )


_tc_call = pl.pallas_call(
    _tc_body,
    grid=(_TOK // _MT,),
    in_specs=[
        pl.BlockSpec((_MT, _NUM_VQ + 2), lambda i: (i, 0)),
        pl.BlockSpec((_NUM_VQ + 2, _D), lambda i: (0, 0)),
        pl.BlockSpec((1, _D), lambda i: (0, 0)),
        pl.BlockSpec((_BS, _CLIP), lambda i: (0, 0)),
        pl.BlockSpec((_CLIP, _D), lambda i: (0, 0)),
        pl.BlockSpec((1, _D), lambda i: (0, 0)),
        pl.BlockSpec((_BS, 1), lambda i: (0, 0)),
        pl.BlockSpec((_NQ, _D), lambda i: (0, 0)),
        pl.BlockSpec((1, _D), lambda i: (0, 0)),
        pl.BlockSpec((_MT, _NQ), lambda i: (jnp.minimum(i, _F // _MT - 1), 0)),
        pl.BlockSpec((_MT, 1), lambda i: (jnp.minimum(i, _F // _MT - 1), 0)),
        pl.BlockSpec((_ROWS, _D), lambda i: (0, 0)),
    ],
    out_specs=[
        pl.BlockSpec((_MT, _D), lambda i: (i, 0)),
        pl.BlockSpec((_BS, _D), lambda i: (0, 0)),
        pl.BlockSpec((_BS, _D), lambda i: (0, 0)),
        pl.BlockSpec((_MT, _D), lambda i: (jnp.minimum(i, _F // _MT - 1), 0)),
    ],
    out_shape=[
        jax.ShapeDtypeStruct((_TOK, _D), jnp.float32),
        jax.ShapeDtypeStruct((_BS, _D), jnp.float32),
        jax.ShapeDtypeStruct((_BS, _D), jnp.float32),
        jax.ShapeDtypeStruct((_F, _D), jnp.float32),
    ],
)


def kernel(clip_feature, q_ids, p_codes, r_codes, W_cond, b_cond, W_pose,
           b_pose, W_quant, b_quant, token_embed_weight):
    qi = q_ids.astype(jnp.int32)
    table = jnp.concatenate(
        [token_embed_weight.reshape(_ROWS, _D),
         jnp.zeros((1, _D), jnp.float32)], axis=0)
    r2 = r_codes.astype(jnp.int32).reshape(_TOK, _NQ)
    rfl = r2[_F:].reshape(_NW, _TPW, _NQ)

    hist_sc = _sc_hist(table, rfl, qi)

    tab16 = token_embed_weight.reshape(_ROWS, _D).astype(jnp.bfloat16)
    qn_tok = lax.rem(jnp.repeat(qi, _T) + (_QM1 - 1), _QM1)[:_F].reshape(_F, 1)
    pemb, cond, qemb, hist_tc = _tc_call(
        p_codes.reshape(_TOK, _NUM_VQ + 2), W_pose, b_pose.reshape(1, _D),
        clip_feature, W_cond, b_cond.reshape(1, _D),
        qi.reshape(_BS, 1), W_quant, b_quant.reshape(1, _D),
        r2[:_F], qn_tok, tab16)
    hist = jnp.concatenate([hist_tc, hist_sc], axis=0)

    return (cond, qemb, pemb.reshape(_BS, _T, _D), hist.reshape(_BS, _T, _D))


# speedup vs baseline: 3.1053x; 1.0002x over previous
import functools

import jax
import jax.numpy as jnp
from jax import lax
from jax.experimental import pallas as pl
from jax.experimental.pallas import tpu as pltpu
from jax.experimental.pallas import tpu_sc as plsc

_NUM_VQ = 512
_NUM_RVQ = 512
_CLIP = 512
_D = 1024
_NQ = 6
_BS = 64
_T = 64

_QM1 = _NQ - 1
_CB = _NUM_RVQ + 2
_ROWS = _QM1 * _CB
_ZROW = _ROWS
_TOK = _BS * _T
_F = 3584
_SCT = _TOK - _F
_NW = 32
_TPW = _SCT // _NW
_G = 8
_RPC = _G * _QM1
_NCHUNK = _TPW // _G
_NIDX = _TPW * _QM1


def _sc_hist_body(table, rfl, qids, hist, r_v, q_v, idx_v, rows_v, rows_v1,
                  out_v, out_v1, sem):
    wid = lax.axis_index("s") * 2 + lax.axis_index("c")
    base = wid * _TPW

    pltpu.sync_copy(rfl.at[wid], r_v)
    pltpu.sync_copy(qids, q_v)

    def idx_step(c, carry):
        p = c * 16 + lax.iota(jnp.int32, 16)
        ch = p // _RPC
        g = p - ch * _RPC
        qq = g >> 3
        t_loc = ch * _G + (g & (_G - 1))
        b = (_F + base + t_loc) >> 6
        rv = plsc.load_gather(r_v, [t_loc, qq])
        qid = plsc.load_gather(q_v, [b])
        n = lax.rem(qid + (_QM1 - 1), _QM1)
        flat = jnp.where(qq <= n, qq * _CB + rv, _ZROW)
        idx_v[pl.ds(c * 16, 16)] = flat
        return carry

    lax.fori_loop(0, _NIDX // 16, idx_step, 0)

    def start_gather(c, buf):
        off = pl.multiple_of(c * _RPC, 8)
        return pltpu.async_copy(table.at[idx_v.at[pl.ds(off, _RPC)]], buf, sem)

    def wait_gather(buf):
        pltpu.make_async_copy(table.at[idx_v.at[pl.ds(0, _RPC)]], buf, sem).wait()

    def accum(rows, out_ref):
        def dstep(dd, carry2):
            sl = pl.ds(dd * 16, 16)
            for t in range(_G):
                s = rows[0 * _G + t, sl] + rows[1 * _G + t, sl]
                s = s + rows[2 * _G + t, sl]
                s = s + rows[3 * _G + t, sl]
                s = s + rows[4 * _G + t, sl]
                out_ref[t, sl] = s
            return carry2
        lax.fori_loop(0, _D // 16, dstep, 0)

    start_gather(0, rows_v)

    def pair_step(i, carry):
        c0 = i * 2
        start_gather(c0 + 1, rows_v1)
        wait_gather(rows_v)
        accum(rows_v, out_v)
        pltpu.sync_copy(out_v, hist.at[pl.ds(base + c0 * _G, _G)])

        @pl.when(i < _NCHUNK // 2 - 1)
        def _():
            start_gather(c0 + 2, rows_v)
        wait_gather(rows_v1)
        accum(rows_v1, out_v1)
        pltpu.sync_copy(out_v1, hist.at[pl.ds(base + (c0 + 1) * _G, _G)])
        return carry

    lax.fori_loop(0, _NCHUNK // 2, pair_step, 0)


_sc_hist = pl.kernel(
    _sc_hist_body,
    out_type=jax.ShapeDtypeStruct((_SCT, _D), jnp.float32),
    compiler_params=pltpu.CompilerParams(needs_layout_passes=False),
    mesh=plsc.VectorSubcoreMesh(core_axis_name="c", subcore_axis_name="s",
                                num_cores=2, num_subcores=16),
    scratch_types=[
        pltpu.VMEM((_TPW, _NQ), jnp.int32),
        pltpu.VMEM((_BS,), jnp.int32),
        pltpu.VMEM((_NIDX,), jnp.int32),
        pltpu.VMEM((_RPC, _D), jnp.float32),
        pltpu.VMEM((_RPC, _D), jnp.float32),
        pltpu.VMEM((_G, _D), jnp.float32),
        pltpu.VMEM((_G, _D), jnp.float32),
        pltpu.SemaphoreType.DMA,
    ],
)

_MT = 256


def _tc_body(p_ref, wpose_ref, bpose_ref, clip_ref, wcond_ref, bcond_ref,
             qi_ref, wq_ref, bq_ref, r_ref, qn_ref, tab_ref,
             pemb_ref, cond_ref, qemb_ref, htc_ref):
    pemb_ref[...] = (
        jnp.dot(p_ref[...].astype(jnp.bfloat16),
                wpose_ref[...].astype(jnp.bfloat16),
                preferred_element_type=jnp.float32)
        + bpose_ref[...]
    )

    @pl.when(pl.program_id(0) < _F // _MT)
    def _():
        acc = None
        col = lax.broadcasted_iota(jnp.int32, (_MT, _CB), 1)
        n = qn_ref[...]
        for q in range(_QM1):
            rq = r_ref[:, q:q + 1]
            oh = ((col == rq).astype(jnp.float32)
                  * (q <= n).astype(jnp.float32)).astype(jnp.bfloat16)
            part = jnp.dot(oh, tab_ref[pl.ds(q * _CB, _CB), :],
                           preferred_element_type=jnp.float32)
            acc = part if acc is None else acc + part
        htc_ref[...] = acc

    @pl.when(pl.program_id(0) == 0)
    def _():
        cond_ref[...] = (
            jnp.dot(clip_ref[...], wcond_ref[...],
                    preferred_element_type=jnp.float32)
            + bcond_ref[...]
        )
        ioq = lax.broadcasted_iota(jnp.int32, (_BS, _NQ), 1)
        oh = (ioq == qi_ref[...]).astype(jnp.float32)
        qemb_ref[...] = (
            jnp.dot(oh, wq_ref[...], preferred_element_type=jnp.float32)
            + bq_ref[...]
        )


_tc_call = pl.pallas_call(
    _tc_body,
    grid=(_TOK // _MT,),
    in_specs=[
        pl.BlockSpec((_MT, _NUM_VQ + 2), lambda i: (i, 0)),
        pl.BlockSpec((_NUM_VQ + 2, _D), lambda i: (0, 0)),
        pl.BlockSpec((1, _D), lambda i: (0, 0)),
        pl.BlockSpec((_BS, _CLIP), lambda i: (0, 0)),
        pl.BlockSpec((_CLIP, _D), lambda i: (0, 0)),
        pl.BlockSpec((1, _D), lambda i: (0, 0)),
        pl.BlockSpec((_BS, 1), lambda i: (0, 0)),
        pl.BlockSpec((_NQ, _D), lambda i: (0, 0)),
        pl.BlockSpec((1, _D), lambda i: (0, 0)),
        pl.BlockSpec((_MT, _NQ), lambda i: (jnp.minimum(i, _F // _MT - 1), 0)),
        pl.BlockSpec((_MT, 1), lambda i: (jnp.minimum(i, _F // _MT - 1), 0)),
        pl.BlockSpec((_ROWS, _D), lambda i: (0, 0)),
    ],
    out_specs=[
        pl.BlockSpec((_MT, _D), lambda i: (i, 0)),
        pl.BlockSpec((_BS, _D), lambda i: (0, 0)),
        pl.BlockSpec((_BS, _D), lambda i: (0, 0)),
        pl.BlockSpec((_MT, _D), lambda i: (jnp.minimum(i, _F // _MT - 1), 0)),
    ],
    out_shape=[
        jax.ShapeDtypeStruct((_TOK, _D), jnp.float32),
        jax.ShapeDtypeStruct((_BS, _D), jnp.float32),
        jax.ShapeDtypeStruct((_BS, _D), jnp.float32),
        jax.ShapeDtypeStruct((_F, _D), jnp.float32),
    ],
)


def kernel(clip_feature, q_ids, p_codes, r_codes, W_cond, b_cond, W_pose,
           b_pose, W_quant, b_quant, token_embed_weight):
    qi = q_ids.astype(jnp.int32)
    table = jnp.concatenate(
        [token_embed_weight.reshape(_ROWS, _D),
         jnp.zeros((1, _D), jnp.float32)], axis=0)
    r2 = r_codes.astype(jnp.int32).reshape(_TOK, _NQ)
    rfl = r2[_F:].reshape(_NW, _TPW, _NQ)

    hist_sc = _sc_hist(table, rfl, qi)

    tab16 = token_embed_weight.reshape(_ROWS, _D).astype(jnp.bfloat16)
    qn_tok = lax.rem(jnp.repeat(qi, _T) + (_QM1 - 1), _QM1)[:_F].reshape(_F, 1)
    pemb, cond, qemb, hist_tc = _tc_call(
        p_codes.reshape(_TOK, _NUM_VQ + 2), W_pose, b_pose.reshape(1, _D),
        clip_feature, W_cond, b_cond.reshape(1, _D),
        qi.reshape(_BS, 1), W_quant, b_quant.reshape(1, _D),
        r2[:_F], qn_tok, tab16)
    hist = jnp.concatenate([hist_tc, hist_sc], axis=0)

    return (cond, qemb, pemb.reshape(_BS, _T, _D), hist.reshape(_BS, _T, _D))


# speedup vs baseline: 3.2257x; 1.0388x over previous
import functools

import jax
import jax.numpy as jnp
from jax import lax
from jax.experimental import pallas as pl
from jax.experimental.pallas import tpu as pltpu
from jax.experimental.pallas import tpu_sc as plsc

_NUM_VQ = 512
_NUM_RVQ = 512
_CLIP = 512
_D = 1024
_NQ = 6
_BS = 64
_T = 64

_QM1 = _NQ - 1
_CB = _NUM_RVQ + 2
_ROWS = _QM1 * _CB
_ZROW = _ROWS
_TOK = _BS * _T
_F = 3840
_SCT = _TOK - _F
_NW = 32
_TPW = _SCT // _NW
_G = 8
_RPC = _G * _QM1
_NCHUNK = _TPW // _G
_NIDX = _TPW * _QM1


def _sc_hist_body(table, rfl, qids, hist, r_v, q_v, idx_v, rows_v, rows_v1,
                  out_v, out_v1, sem):
    wid = lax.axis_index("s") * 2 + lax.axis_index("c")
    base = wid * _TPW

    pltpu.sync_copy(rfl.at[wid], r_v)
    pltpu.sync_copy(qids, q_v)

    def idx_step(c, carry):
        p = jnp.minimum(c * 16 + lax.iota(jnp.int32, 16), _NIDX - 1)
        ch = p // _RPC
        g = p - ch * _RPC
        qq = g >> 3
        t_loc = ch * _G + (g & (_G - 1))
        b = (_F + base + t_loc) >> 6
        rv = plsc.load_gather(r_v, [t_loc, qq])
        qid = plsc.load_gather(q_v, [b])
        n = lax.rem(qid + (_QM1 - 1), _QM1)
        flat = jnp.where(qq <= n, qq * _CB + rv, _ZROW)
        idx_v[pl.ds(c * 16, 16)] = flat
        return carry

    lax.fori_loop(0, (_NIDX + 15) // 16, idx_step, 0)

    def start_gather(c, buf):
        off = pl.multiple_of(c * _RPC, 8)
        return pltpu.async_copy(table.at[idx_v.at[pl.ds(off, _RPC)]], buf, sem)

    def wait_gather(buf):
        pltpu.make_async_copy(table.at[idx_v.at[pl.ds(0, _RPC)]], buf, sem).wait()

    def accum(rows, out_ref):
        def dstep(dd, carry2):
            sl = pl.ds(dd * 16, 16)
            for t in range(_G):
                s = rows[0 * _G + t, sl] + rows[1 * _G + t, sl]
                s = s + rows[2 * _G + t, sl]
                s = s + rows[3 * _G + t, sl]
                s = s + rows[4 * _G + t, sl]
                out_ref[t, sl] = s
            return carry2
        lax.fori_loop(0, _D // 16, dstep, 0)

    start_gather(0, rows_v)

    if _NCHUNK == 1:
        wait_gather(rows_v)
        accum(rows_v, out_v)
        pltpu.sync_copy(out_v, hist.at[pl.ds(base, _G)])
    else:
        def pair_step(i, carry):
            c0 = i * 2
            start_gather(c0 + 1, rows_v1)
            wait_gather(rows_v)
            accum(rows_v, out_v)
            pltpu.sync_copy(out_v, hist.at[pl.ds(base + c0 * _G, _G)])

            @pl.when(i < _NCHUNK // 2 - 1)
            def _():
                start_gather(c0 + 2, rows_v)
            wait_gather(rows_v1)
            accum(rows_v1, out_v1)
            pltpu.sync_copy(out_v1, hist.at[pl.ds(base + (c0 + 1) * _G, _G)])
            return carry

        lax.fori_loop(0, _NCHUNK // 2, pair_step, 0)


_sc_hist = pl.kernel(
    _sc_hist_body,
    out_type=jax.ShapeDtypeStruct((_SCT, _D), jnp.float32),
    compiler_params=pltpu.CompilerParams(needs_layout_passes=False),
    mesh=plsc.VectorSubcoreMesh(core_axis_name="c", subcore_axis_name="s",
                                num_cores=2, num_subcores=16),
    scratch_types=[
        pltpu.VMEM((_TPW, _NQ), jnp.int32),
        pltpu.VMEM((_BS,), jnp.int32),
        pltpu.VMEM(((_NIDX + 15) // 16 * 16,), jnp.int32),
        pltpu.VMEM((_RPC, _D), jnp.float32),
        pltpu.VMEM((_RPC, _D), jnp.float32),
        pltpu.VMEM((_G, _D), jnp.float32),
        pltpu.VMEM((_G, _D), jnp.float32),
        pltpu.SemaphoreType.DMA,
    ],
)

_MT = 256


def _tc_body(p_ref, wpose_ref, bpose_ref, clip_ref, wcond_ref, bcond_ref,
             qi_ref, wq_ref, bq_ref, r_ref, qn_ref, tab_ref,
             pemb_ref, cond_ref, qemb_ref, htc_ref):
    pemb_ref[...] = (
        jnp.dot(p_ref[...].astype(jnp.bfloat16),
                wpose_ref[...].astype(jnp.bfloat16),
                preferred_element_type=jnp.float32)
        + bpose_ref[...]
    )

    @pl.when(pl.program_id(0) < _F // _MT)
    def _():
        acc = None
        col = lax.broadcasted_iota(jnp.int32, (_MT, _CB), 1)
        n = qn_ref[...]
        for q in range(_QM1):
            rq = r_ref[:, q:q + 1]
            oh = ((col == rq).astype(jnp.float32)
                  * (q <= n).astype(jnp.float32)).astype(jnp.bfloat16)
            part = jnp.dot(oh, tab_ref[pl.ds(q * _CB, _CB), :],
                           preferred_element_type=jnp.float32)
            acc = part if acc is None else acc + part
        htc_ref[...] = acc

    @pl.when(pl.program_id(0) == 0)
    def _():
        cond_ref[...] = (
            jnp.dot(clip_ref[...], wcond_ref[...],
                    preferred_element_type=jnp.float32)
            + bcond_ref[...]
        )
        ioq = lax.broadcasted_iota(jnp.int32, (_BS, _NQ), 1)
        oh = (ioq == qi_ref[...]).astype(jnp.float32)
        qemb_ref[...] = (
            jnp.dot(oh, wq_ref[...], preferred_element_type=jnp.float32)
            + bq_ref[...]
        )


_tc_call = pl.pallas_call(
    _tc_body,
    grid=(_TOK // _MT,),
    in_specs=[
        pl.BlockSpec((_MT, _NUM_VQ + 2), lambda i: (i, 0)),
        pl.BlockSpec((_NUM_VQ + 2, _D), lambda i: (0, 0)),
        pl.BlockSpec((1, _D), lambda i: (0, 0)),
        pl.BlockSpec((_BS, _CLIP), lambda i: (0, 0)),
        pl.BlockSpec((_CLIP, _D), lambda i: (0, 0)),
        pl.BlockSpec((1, _D), lambda i: (0, 0)),
        pl.BlockSpec((_BS, 1), lambda i: (0, 0)),
        pl.BlockSpec((_NQ, _D), lambda i: (0, 0)),
        pl.BlockSpec((1, _D), lambda i: (0, 0)),
        pl.BlockSpec((_MT, _NQ), lambda i: (jnp.minimum(i, _F // _MT - 1), 0)),
        pl.BlockSpec((_MT, 1), lambda i: (jnp.minimum(i, _F // _MT - 1), 0)),
        pl.BlockSpec((_ROWS, _D), lambda i: (0, 0)),
    ],
    out_specs=[
        pl.BlockSpec((_MT, _D), lambda i: (i, 0)),
        pl.BlockSpec((_BS, _D), lambda i: (0, 0)),
        pl.BlockSpec((_BS, _D), lambda i: (0, 0)),
        pl.BlockSpec((_MT, _D), lambda i: (jnp.minimum(i, _F // _MT - 1), 0)),
    ],
    out_shape=[
        jax.ShapeDtypeStruct((_TOK, _D), jnp.float32),
        jax.ShapeDtypeStruct((_BS, _D), jnp.float32),
        jax.ShapeDtypeStruct((_BS, _D), jnp.float32),
        jax.ShapeDtypeStruct((_F, _D), jnp.float32),
    ],
)


def kernel(clip_feature, q_ids, p_codes, r_codes, W_cond, b_cond, W_pose,
           b_pose, W_quant, b_quant, token_embed_weight):
    qi = q_ids.astype(jnp.int32)
    table = jnp.concatenate(
        [token_embed_weight.reshape(_ROWS, _D),
         jnp.zeros((1, _D), jnp.float32)], axis=0)
    r2 = r_codes.astype(jnp.int32).reshape(_TOK, _NQ)
    rfl = r2[_F:].reshape(_NW, _TPW, _NQ)

    hist_sc = _sc_hist(table, rfl, qi)

    tab16 = token_embed_weight.reshape(_ROWS, _D).astype(jnp.bfloat16)
    qn_tok = lax.rem(jnp.repeat(qi, _T) + (_QM1 - 1), _QM1)[:_F].reshape(_F, 1)
    pemb, cond, qemb, hist_tc = _tc_call(
        p_codes.reshape(_TOK, _NUM_VQ + 2), W_pose, b_pose.reshape(1, _D),
        clip_feature, W_cond, b_cond.reshape(1, _D),
        qi.reshape(_BS, 1), W_quant, b_quant.reshape(1, _D),
        r2[:_F], qn_tok, tab16)
    hist = jnp.concatenate([hist_tc, hist_sc], axis=0)

    return (cond, qemb, pemb.reshape(_BS, _T, _D), hist.reshape(_BS, _T, _D))
